# baseline (speedup 1.0000x reference)
import functools

import jax
import jax.numpy as jnp
from jax.experimental import pallas as pl
from jax.experimental.pallas import tpu as pltpu


def _sig(x):
    return 0.5 + 0.5 * jnp.tanh(0.5 * x)


_AI, _AF, _AG, _AO = 0, 8, 16, 24


def _acc_chain(h_bf, m, wh, hidden):
    pltpu.matmul_push_rhs(wh[:, 2 * hidden:3 * hidden], 0, m)
    pltpu.matmul_push_rhs(wh[:, :hidden], 1, m)
    pltpu.matmul_acc_lhs(_AG, h_bf, m, load_staged_rhs=0)
    pltpu.matmul_acc_lhs(_AI, h_bf, m, load_staged_rhs=1)
    pltpu.matmul_push_rhs(wh[:, hidden:2 * hidden], 0, m)
    pltpu.matmul_push_rhs(wh[:, 3 * hidden:], 1, m)
    pltpu.matmul_acc_lhs(_AF, h_bf, m, load_staged_rhs=0)
    pltpu.matmul_acc_lhs(_AO, h_bf, m, load_staged_rhs=1)


def _pop_act(m, rows, gx_ref, r0, c, hidden):
    gg = pltpu.matmul_pop(_AG, (rows, hidden), jnp.float32, m)
    gi = pltpu.matmul_pop(_AI, (rows, hidden), jnp.float32, m)
    gf = pltpu.matmul_pop(_AF, (rows, hidden), jnp.float32, m)
    go = pltpu.matmul_pop(_AO, (rows, hidden), jnp.float32, m)
    gi = gi + gx_ref[r0:r0 + rows, :hidden]
    gf = gf + gx_ref[r0:r0 + rows, hidden:2 * hidden]
    gg = gg + gx_ref[r0:r0 + rows, 2 * hidden:3 * hidden]
    go = go + gx_ref[r0:r0 + rows, 3 * hidden:]
    c = _sig(gf) * c + _sig(gi) * jnp.tanh(gg)
    h = _sig(go) * jnp.tanh(c)
    return h, c


def _lstm_body(x_ref, wxp_ref, wh_ref, wfp_ref, bf_ref,
               out_ref, h_ref, c_ref, gx_ref, *, nchunks, chunk, bc, hidden,
               out_dim):
    k = pl.program_id(1)
    half = bc // 2
    rows = chunk * bc
    hrows = rows // 2

    @pl.when(k == 0)
    def _():
        h_ref[...] = jnp.zeros_like(h_ref)
        c_ref[...] = jnp.zeros_like(c_ref)

    wxp = wxp_ref[...]
    xh0 = x_ref[0, 0, :hrows, :]
    xh1 = x_ref[0, 0, hrows:, :]
    xc0 = jnp.concatenate([xh0, xh0], axis=-1)
    xc1 = jnp.concatenate([xh1, xh1], axis=-1)
    for m in (0, 1):
        pltpu.matmul_push_rhs(wxp[:, 2 * m * hidden:(2 * m + 1) * hidden],
                              0, m)
        pltpu.matmul_push_rhs(wxp[:, (2 * m + 1) * hidden:(2 * m + 2) * hidden],
                              1, m)
    for m in (0, 1):
        pltpu.matmul_acc_lhs(0, xc0, m, load_staged_rhs=0)
        pltpu.matmul_acc_lhs(128, xc1, m, load_staged_rhs=None)
    for m in (0, 1):
        t = 2 * m
        gx_ref[:hrows, t * hidden:(t + 1) * hidden] = (
            pltpu.matmul_pop(0, (hrows, hidden), jnp.float32, m))
        gx_ref[hrows:, t * hidden:(t + 1) * hidden] = (
            pltpu.matmul_pop(128, (hrows, hidden), jnp.float32, m))
    for m in (0, 1):
        pltpu.matmul_acc_lhs(0, xc0, m, load_staged_rhs=1)
        pltpu.matmul_acc_lhs(128, xc1, m, load_staged_rhs=None)
    for m in (0, 1):
        t = 2 * m + 1
        gx_ref[:hrows, t * hidden:(t + 1) * hidden] = (
            pltpu.matmul_pop(0, (hrows, hidden), jnp.float32, m))
        gx_ref[hrows:, t * hidden:(t + 1) * hidden] = (
            pltpu.matmul_pop(128, (hrows, hidden), jnp.float32, m))

    wh = wh_ref[...]
    ha = h_ref[:half, :]
    ca = c_ref[:half, :]
    hb = h_ref[half:, :]
    cb = c_ref[half:, :]

    _acc_chain(ha.astype(jnp.bfloat16), 0, wh, hidden)
    _acc_chain(hb.astype(jnp.bfloat16), 1, wh, hidden)
    for j in range(chunk):
        ha, ca = _pop_act(0, half, gx_ref, j * bc, ca, hidden)
        if j + 1 < chunk:
            _acc_chain(ha.astype(jnp.bfloat16), 0, wh, hidden)
        hb, cb = _pop_act(1, half, gx_ref, j * bc + half, cb, hidden)
        if j + 1 < chunk:
            _acc_chain(hb.astype(jnp.bfloat16), 1, wh, hidden)
    h_ref[:half, :] = ha
    c_ref[:half, :] = ca
    h_ref[half:, :] = hb
    c_ref[half:, :] = cb

    pltpu.matmul_push_rhs(wfp_ref[...], 0, 0)
    hcat = jnp.concatenate([ha, hb], axis=0).astype(jnp.bfloat16)
    pltpu.matmul_acc_lhs(0, hcat, 0, load_staged_rhs=0)
    res = pltpu.matmul_pop(0, (bc, hidden), jnp.float32, 0)
    out_ref[...] = res[:, :out_dim] + bf_ref[...]


def kernel(x, input_lengths, W_ih, W_hh, b_ih, b_hh, W_xi, b_xi, W_fc, b_fc):
    del input_lengths, W_xi, b_xi
    B, T, IN = x.shape
    H = W_hh.shape[1]
    OUT = W_fc.shape[0]
    NC = 2
    Bc = B // NC
    CHUNK = 16 if T % 16 == 0 else 1
    K = T // CHUNK
    LANES = 128

    xr = (jnp.swapaxes(x, 0, 1)
          .reshape(K, CHUNK, NC, Bc, IN)
          .transpose(2, 0, 1, 3, 4)
          .reshape(NC, K, CHUNK * Bc, IN))
    pad = jnp.zeros((NC, K, CHUNK * Bc, LANES - IN - 1), x.dtype)
    ones = jnp.ones((NC, K, CHUNK * Bc, 1), x.dtype)
    xr = jnp.concatenate([xr, ones, pad], axis=-1).astype(jnp.bfloat16)

    Wxp = jnp.zeros((H, 4 * H), jnp.float32)
    Wxp = Wxp.at[:IN, :].set(W_ih[:, :IN].T)
    Wxp = Wxp.at[IN, :].set(b_ih + b_hh)
    Wxp = Wxp.astype(jnp.bfloat16)

    Wh = W_hh.T.astype(jnp.bfloat16)
    Wfp = jnp.concatenate(
        [W_fc.T, jnp.zeros((H, H - OUT), jnp.float32)],
        axis=1).astype(jnp.bfloat16)
    bf = b_fc[None, :]

    body = functools.partial(_lstm_body, nchunks=K, chunk=CHUNK, bc=Bc,
                             hidden=H, out_dim=OUT)

    out = pl.pallas_call(
        body,
        grid=(NC, K),
        in_specs=[
            pl.BlockSpec((1, 1, CHUNK * Bc, LANES), lambda n, k: (n, k, 0, 0)),
            pl.BlockSpec((H, 4 * H), lambda n, k: (0, 0)),
            pl.BlockSpec((H, 4 * H), lambda n, k: (0, 0)),
            pl.BlockSpec((H, H), lambda n, k: (0, 0)),
            pl.BlockSpec((1, OUT), lambda n, k: (0, 0)),
        ],
        out_specs=pl.BlockSpec((Bc, OUT), lambda n, k: (n, 0)),
        out_shape=jax.ShapeDtypeStruct((B, OUT), jnp.float32),
        scratch_shapes=[
            pltpu.VMEM((Bc, H), jnp.float32),
            pltpu.VMEM((Bc, H), jnp.float32),
            pltpu.VMEM((CHUNK * Bc, 4 * H), jnp.float32),
        ],
        compiler_params=pltpu.CompilerParams(
            dimension_semantics=("parallel", "arbitrary")),
    )(xr, Wxp, Wh, Wfp, bf)
    return out


# speedup vs baseline: 1.0877x; 1.0877x over previous
import functools

import jax
import jax.numpy as jnp
from jax.experimental import pallas as pl
from jax.experimental.pallas import tpu as pltpu


def _sig(x):
    return 0.5 + 0.5 * jnp.tanh(0.5 * x)


def _lstm_body(x_ref, wx_ref, wh_ref, wf_ref, bf_ref,
               out_ref, h_ref, c_ref, gx_ref, *, nchunks, chunk, bc, hidden):
    k = pl.program_id(1)

    @pl.when(k == 0)
    def _():
        h_ref[...] = jnp.zeros_like(h_ref)
        c_ref[...] = jnp.zeros_like(c_ref)

    gx_ref[...] = jnp.dot(x_ref[0, 0], wx_ref[...],
                          preferred_element_type=jnp.float32)

    half = bc // 2
    wh = wh_ref[...]

    def act(gates, c):
        i = gates[:, :hidden]
        f = gates[:, hidden:2 * hidden]
        g = gates[:, 2 * hidden:3 * hidden]
        o = gates[:, 3 * hidden:]
        c = _sig(f) * c + _sig(i) * jnp.tanh(g)
        return _sig(o) * jnp.tanh(c), c

    ha = h_ref[:half, :]
    ca = c_ref[:half, :]
    hb = h_ref[half:, :]
    cb = c_ref[half:, :]
    for j in range(chunk):
        ga = gx_ref[j * bc:j * bc + half, :] + jnp.dot(
            ha.astype(jnp.bfloat16), wh, preferred_element_type=jnp.float32)
        ha, ca = act(ga, ca)
        gb = gx_ref[j * bc + half:(j + 1) * bc, :] + jnp.dot(
            hb.astype(jnp.bfloat16), wh, preferred_element_type=jnp.float32)
        hb, cb = act(gb, cb)
    h_ref[:half, :] = ha
    c_ref[:half, :] = ca
    h_ref[half:, :] = hb
    c_ref[half:, :] = cb

    @pl.when(k == nchunks - 1)
    def _():
        h = jnp.concatenate([ha, hb], axis=0)
        out_ref[...] = (jnp.dot(h, wf_ref[...],
                                preferred_element_type=jnp.float32)
                        + bf_ref[...])


def kernel(x, input_lengths, W_ih, W_hh, b_ih, b_hh, W_xi, b_xi, W_fc, b_fc):
    del input_lengths, W_xi, b_xi
    B, T, IN = x.shape
    H = W_hh.shape[1]
    OUT = W_fc.shape[0]
    NC = 2
    Bc = B // NC
    CHUNK = 64 if T % 64 == 0 else 1
    K = T // CHUNK

    xr = (jnp.swapaxes(x, 0, 1)
          .reshape(K, CHUNK, NC, Bc, IN)
          .transpose(2, 0, 1, 3, 4)
          .reshape(NC, K, CHUNK * Bc, IN))
    ones = jnp.ones((NC, K, CHUNK * Bc, 1), x.dtype)
    xr = jnp.concatenate([xr, ones], axis=-1)
    Wx = jnp.concatenate([W_ih[:, :IN].T, (b_ih + b_hh)[None, :]], axis=0)
    Wh = W_hh.T.astype(jnp.bfloat16)
    Wf = W_fc.T
    bf = b_fc[None, :]

    body = functools.partial(_lstm_body, nchunks=K, chunk=CHUNK, bc=Bc,
                             hidden=H)

    out = pl.pallas_call(
        body,
        grid=(NC, K),
        in_specs=[
            pl.BlockSpec((1, 1, CHUNK * Bc, IN + 1), lambda n, k: (n, k, 0, 0)),
            pl.BlockSpec((IN + 1, 4 * H), lambda n, k: (0, 0)),
            pl.BlockSpec((H, 4 * H), lambda n, k: (0, 0)),
            pl.BlockSpec((H, OUT), lambda n, k: (0, 0)),
            pl.BlockSpec((1, OUT), lambda n, k: (0, 0)),
        ],
        out_specs=pl.BlockSpec((Bc, OUT), lambda n, k: (n, 0)),
        out_shape=jax.ShapeDtypeStruct((B, OUT), jnp.float32),
        scratch_shapes=[
            pltpu.VMEM((Bc, H), jnp.float32),
            pltpu.VMEM((Bc, H), jnp.float32),
            pltpu.VMEM((CHUNK * Bc, 4 * H), jnp.float32),
        ],
        compiler_params=pltpu.CompilerParams(
            dimension_semantics=("parallel", "arbitrary")),
    )(xr, Wx, Wh, Wf, bf)
    return out
